# Initial kernel scaffold; baseline (speedup 1.0000x reference)
import jax
import jax.numpy as jnp
from jax.experimental import pallas as pl


def kernel(x, edge_index, edge_attr, params):
    raise NotImplementedError("write your pallas kernel here")


# speedup vs baseline: 10.8117x; 10.8117x over previous
import functools
import numpy as np
import jax
import jax.numpy as jnp
from jax.experimental import pallas as pl
from jax.experimental.pallas import tpu as pltpu

F_IN, T, F_OUT, H = 100, 5, 20, 100
ALOG = float(np.log(17.0))
F5 = T * F_IN
F5P = 512
HP = 128


def _pad_to(a, n, axis):
    pad = n - a.shape[axis]
    if pad <= 0:
        return a
    cfg = [(0, 0)] * a.ndim
    cfg[axis] = (0, pad)
    return jnp.pad(a, cfg)


def _mm_body(x_ref, w_ref, b_ref, o_ref, *, relu):
    acc = jnp.dot(x_ref[...], w_ref[...], preferred_element_type=jnp.float32)
    acc = acc + b_ref[...]
    if relu:
        acc = jnp.maximum(acc, 0.0)
    o_ref[...] = acc


def _matmul(x, w, b, relu=False, bm=512):
    M, K = x.shape
    Nc = w.shape[1]
    assert M % bm == 0, (M, bm)
    grid = (M // bm,)
    return pl.pallas_call(
        functools.partial(_mm_body, relu=relu),
        grid=grid,
        in_specs=[
            pl.BlockSpec((bm, K), lambda i: (i, 0)),
            pl.BlockSpec((K, Nc), lambda i: (0, 0)),
            pl.BlockSpec((1, Nc), lambda i: (0, 0)),
        ],
        out_specs=pl.BlockSpec((bm, Nc), lambda i: (i, 0)),
        out_shape=jax.ShapeDtypeStruct((M, Nc), jnp.float32),
    )(x, w, b.reshape(1, Nc))


def kernel(x, edge_index, edge_attr, params):
    N = x.shape[0]
    E = edge_index.shape[1]
    src, dst = edge_index[0], edge_index[1]

    perm = jnp.argsort(dst)
    src_s = src[perm].astype(jnp.int32)
    dst_s = dst[perm].astype(jnp.int32)
    ea_s = edge_attr[perm]
    row_start = jnp.searchsorted(dst_s, jnp.arange(N + 1), side='left').astype(jnp.int32)
    cnt = (row_start[1:] - row_start[:-1]).astype(jnp.float32)

    p = params
    node_W = _pad_to(p['node_W'], HP, 1)
    node_b = _pad_to(p['node_b'], HP, 0)
    edge_W = _pad_to(_pad_to(p['edge_W'], 32, 0), HP, 1)
    edge_b = _pad_to(p['edge_b'], HP, 0)

    h = _matmul(x, node_W, node_b, bm=400)
    ea_p = _pad_to(ea_s, 32, 1)
    e = _matmul(ea_p, edge_W, edge_b, bm=640)

    for lp in p['layers']:
        Wa = lp['Wpre'][:, :F_IN, :]
        Wb = lp['Wpre'][:, F_IN:2 * F_IN, :]
        Wc = lp['Wpre'][:, 2 * F_IN:, :]
        WeeC = jnp.einsum('hf,tfo->hto', lp['Wee'], Wc).reshape(H, F5)
        b_all = (jnp.einsum('f,tfo->to', lp['bee'], Wc) + lp['bpre']).reshape(F5)
        WaF = jnp.transpose(Wa, (1, 0, 2)).reshape(F_IN, F5)
        WbF = jnp.transpose(Wb, (1, 0, 2)).reshape(F_IN, F5)

        WA = _pad_to(_pad_to(WaF, HP, 0), F5P, 1)
        bA = _pad_to(b_all, F5P, 0)
        A = _matmul(h, WA, bA, bm=400)

        hs = h[src_s]
        WS = _pad_to(jnp.concatenate(
            [_pad_to(WbF, HP, 0), _pad_to(WeeC, HP, 0)], axis=0), F5P, 1)
        S = _matmul(jnp.concatenate([hs, e], axis=1), WS,
                    jnp.zeros((F5P,), jnp.float32), bm=640)

        sumS = jax.ops.segment_sum(S, dst_s, num_segments=N)
        sumsqS = jax.ops.segment_sum(S * S, dst_s, num_segments=N)
        minS = jax.ops.segment_min(S, dst_s, num_segments=N)
        maxS = jax.ops.segment_max(S, dst_s, num_segments=N)

        empty = (cnt == 0)[:, None]
        deg = jnp.clip(cnt, 1.0, None)[:, None]
        meanS = sumS / deg
        mean = jnp.where(empty, 0.0, A + meanS)[:, :F5]
        var = jnp.maximum(sumsqS / deg - meanS * meanS, 0.0)
        std = jnp.sqrt(var + 1e-5)[:, :F5]
        mn = jnp.where(empty, 0.0, A + minS)[:, :F5]
        mx = jnp.where(empty, 0.0, A + maxS)[:, :F5]

        agg = jnp.concatenate(
            [z.reshape(N, T, F_IN) for z in (mean, mn, mx, std)], axis=-1)
        ampl = (jnp.log(deg + 1.0) / ALOG)[:, :, None]
        atten = (ALOG / jnp.log(deg + 1.0))[:, :, None]

        Wp_x = lp['Wpost'][:, :F_IN, :]
        Wp_a = lp['Wpost'][:, F_IN:, :].reshape(T, 3, 4 * F_IN, F_OUT)
        def tower_mm(z, W):
            return jnp.einsum('ntf,tfo->nto', z, W)
        E1 = tower_mm(agg, Wp_a[:, 0])
        E2 = tower_mm(agg, Wp_a[:, 1])
        E3 = tower_mm(agg, Wp_a[:, 2])
        out = (jnp.einsum('nf,tfo->nto', h[:, :H], Wp_x)
               + E1 + E2 * ampl + E3 * atten + lp['bpost'])
        Wlin = _pad_to(_pad_to(lp['Wlin'], HP, 0), HP, 1)
        blin = _pad_to(lp['blin'], HP, 0)
        c = _matmul(out.reshape(N, T * F_OUT), Wlin[:T * F_OUT], blin, bm=400)

        mu = jnp.mean(c, axis=0)
        vv = jnp.mean(c * c, axis=0) - mu * mu
        gam = _pad_to(lp['bn_g'], HP, 0)
        bet = _pad_to(lp['bn_b'], HP, 0)
        cbn = (c - mu) / jnp.sqrt(vv + 1e-5) * gam + bet
        mask = (jnp.arange(HP) < H).astype(jnp.float32)
        h = h + jnp.maximum(cbn, 0.0) * mask / 2.0

        hs2 = h[src_s]
        hd2 = h[dst_s]
        W1 = jnp.concatenate([
            _pad_to(lp['eW1'][:H], HP, 0),
            _pad_to(lp['eW1'][H:2 * H], HP, 0),
            _pad_to(lp['eW1'][2 * H:], HP, 0)], axis=0)
        W1 = _pad_to(W1, HP, 1)
        z = _matmul(jnp.concatenate([hs2, hd2, e], axis=1), W1,
                    _pad_to(lp['eb1'], HP, 0), relu=True, bm=640)
        W2 = _pad_to(_pad_to(lp['eW2'], HP, 0), HP, 1)
        e = e + _matmul(z, W2, _pad_to(lp['eb2'], HP, 0), bm=640) / 2.0

    W1 = _pad_to(_pad_to(p['mlp_W1'], HP, 0), 64, 1)
    o = _matmul(h, W1, _pad_to(p['mlp_b1'], 64, 0), relu=True, bm=400)
    W2 = _pad_to(_pad_to(p['mlp_W2'], 64, 0), 32, 1)
    o = _matmul(o, W2, _pad_to(p['mlp_b2'], 32, 0), relu=True, bm=400)
    W3 = _pad_to(_pad_to(p['mlp_W3'], 32, 0), 8, 1)
    o = _matmul(o, W3, _pad_to(p['mlp_b3'], 8, 0), bm=400)
    return o[:, :2]


# speedup vs baseline: 12.4568x; 1.1522x over previous
import functools
import numpy as np
import jax
import jax.numpy as jnp
from jax import lax
from jax.experimental import pallas as pl
from jax.experimental.pallas import tpu as pltpu
from jax.experimental.pallas import tpu_sc as plsc

F_IN, T, F_OUT, H = 100, 5, 20, 100
ALOG = float(np.log(17.0))
F5 = T * F_IN
F5P = 512
HP = 128

NC_SC, NS_SC = 2, 16
NW = NC_SC * NS_SC
NPW = 313
NP = NW * NPW
CE = 128


def _sc_segment_stats(S, rs2d):
    E = S.shape[0]
    mesh = plsc.VectorSubcoreMesh(core_axis_name="c", subcore_axis_name="s")

    @functools.partial(
        pl.kernel, mesh=mesh,
        out_type=jax.ShapeDtypeStruct((NP, 4, F5P), jnp.float32),
        scratch_types=[
            pltpu.VMEM((1, 320), jnp.int32),
            pltpu.VMEM((CE, F5P), jnp.float32),
            pltpu.VMEM((4, F5P), jnp.float32),
        ],
    )
    def k(S_hbm, rs_hbm, out_hbm, rs_v, buf, acc):
        wid = lax.axis_index("s") * NC_SC + lax.axis_index("c")
        pltpu.sync_copy(rs_hbm.at[wid], rs_v)

        def node_body(n, loaded_q):
            rsw = rs_v[0, pl.ds(n, 16)]
            start = rsw[0]
            end = rsw[1]
            qmax = E // CE - 1
            q0 = jnp.minimum(lax.div(start, CE), qmax)
            q1 = jnp.minimum(lax.div(jnp.maximum(end - 1, start), CE), qmax)

            def chunk_body(q, lq):
                @pl.when(q != lq)
                def _():
                    base = pl.multiple_of(q * CE, CE)
                    pltpu.sync_copy(S_hbm.at[pl.ds(base, CE)], buf)

                lo = jnp.maximum(start, q * CE)
                hi = jnp.minimum(end, (q + 1) * CE)
                first = lo == start
                rel = q * CE
                for c in range(F5P // 16):
                    sl = pl.ds(c * 16, 16)
                    zero = jnp.zeros((16,), jnp.float32)
                    big = jnp.full((16,), 3.0e38, jnp.float32)
                    s0 = jnp.where(first, zero, acc[0, sl])
                    q0v = jnp.where(first, zero, acc[1, sl])
                    mn0 = jnp.where(first, big, acc[2, sl])
                    mx0 = jnp.where(first, -big, acc[3, sl])

                    def ebody(i, st4):
                        s, sq, mn, mx = st4
                        v = buf[i - rel, sl]
                        return (s + v, sq + v * v,
                                jnp.minimum(mn, v), jnp.maximum(mx, v))

                    s, sq, mn, mx = lax.fori_loop(lo, hi, ebody,
                                                  (s0, q0v, mn0, mx0))
                    acc[0, sl] = s
                    acc[1, sl] = sq
                    acc[2, sl] = mn
                    acc[3, sl] = mx

                @pl.when(hi == end)
                def _():
                    pltpu.sync_copy(acc, out_hbm.at[wid * NPW + n])

                return q

            return lax.fori_loop(q0, q1 + 1, chunk_body, loaded_q)

        lax.fori_loop(0, NPW, node_body, jnp.int32(-1))

    return k(S, rs2d)


def _pad_to(a, n, axis):
    pad = n - a.shape[axis]
    if pad <= 0:
        return a
    cfg = [(0, 0)] * a.ndim
    cfg[axis] = (0, pad)
    return jnp.pad(a, cfg)


def _mm_body(x_ref, w_ref, b_ref, o_ref, *, relu):
    acc = jnp.dot(x_ref[...], w_ref[...], preferred_element_type=jnp.float32)
    acc = acc + b_ref[...]
    if relu:
        acc = jnp.maximum(acc, 0.0)
    o_ref[...] = acc


def _matmul(x, w, b, relu=False, bm=512):
    M, K = x.shape
    Nc = w.shape[1]
    assert M % bm == 0, (M, bm)
    grid = (M // bm,)
    return pl.pallas_call(
        functools.partial(_mm_body, relu=relu),
        grid=grid,
        in_specs=[
            pl.BlockSpec((bm, K), lambda i: (i, 0)),
            pl.BlockSpec((K, Nc), lambda i: (0, 0)),
            pl.BlockSpec((1, Nc), lambda i: (0, 0)),
        ],
        out_specs=pl.BlockSpec((bm, Nc), lambda i: (i, 0)),
        out_shape=jax.ShapeDtypeStruct((M, Nc), jnp.float32),
    )(x, w, b.reshape(1, Nc))


def kernel(x, edge_index, edge_attr, params):
    N = x.shape[0]
    E = edge_index.shape[1]
    src, dst = edge_index[0], edge_index[1]

    perm = jnp.argsort(dst)
    src_s = src[perm].astype(jnp.int32)
    dst_s = dst[perm].astype(jnp.int32)
    ea_s = edge_attr[perm]
    row_start = jnp.searchsorted(dst_s, jnp.arange(N + 1), side='left').astype(jnp.int32)
    cnt = (row_start[1:] - row_start[:-1]).astype(jnp.float32)
    rs_pad = jnp.concatenate(
        [row_start, jnp.full((NP + 320 - (N + 1),), E, jnp.int32)])
    rs2d = jnp.stack(
        [lax.dynamic_slice(rs_pad, (w * NPW,), (320,)) for w in range(NW)]
    ).reshape(NW, 1, 320)

    p = params
    node_W = _pad_to(p['node_W'], HP, 1)
    node_b = _pad_to(p['node_b'], HP, 0)
    edge_W = _pad_to(_pad_to(p['edge_W'], 32, 0), HP, 1)
    edge_b = _pad_to(p['edge_b'], HP, 0)

    h = _matmul(x, node_W, node_b, bm=400)
    ea_p = _pad_to(ea_s, 32, 1)
    e = _matmul(ea_p, edge_W, edge_b, bm=640)

    for lp in p['layers']:
        Wa = lp['Wpre'][:, :F_IN, :]
        Wb = lp['Wpre'][:, F_IN:2 * F_IN, :]
        Wc = lp['Wpre'][:, 2 * F_IN:, :]
        WeeC = jnp.einsum('hf,tfo->hto', lp['Wee'], Wc,
                          precision=lax.Precision.HIGHEST).reshape(H, F5)
        b_all = (jnp.einsum('f,tfo->to', lp['bee'], Wc) + lp['bpre']).reshape(F5)
        WaF = jnp.transpose(Wa, (1, 0, 2)).reshape(F_IN, F5)
        WbF = jnp.transpose(Wb, (1, 0, 2)).reshape(F_IN, F5)

        WA = _pad_to(_pad_to(WaF, HP, 0), F5P, 1)
        bA = _pad_to(b_all, F5P, 0)
        A = _matmul(h, WA, bA, bm=400)

        hs = h[src_s]
        WS = _pad_to(jnp.concatenate(
            [_pad_to(WbF, HP, 0), _pad_to(WeeC, HP, 0)], axis=0), F5P, 1)
        S = _matmul(jnp.concatenate([hs, e], axis=1), WS,
                    jnp.zeros((F5P,), jnp.float32), bm=640)

        stats = _sc_segment_stats(S, rs2d)[:N]
        sumS, sumsqS, minS, maxS = (stats[:, i] for i in range(4))

        empty = (cnt == 0)[:, None]
        deg = jnp.clip(cnt, 1.0, None)[:, None]
        meanS = jnp.where(empty, 0.0, sumS / deg)
        mean = jnp.where(empty, 0.0, A + meanS)[:, :F5]
        var = jnp.maximum(jnp.where(empty, 0.0, sumsqS / deg) - meanS * meanS, 0.0)
        std = jnp.sqrt(var + 1e-5)[:, :F5]
        mn = jnp.where(empty, 0.0, A + minS)[:, :F5]
        mx = jnp.where(empty, 0.0, A + maxS)[:, :F5]

        agg = jnp.concatenate(
            [z.reshape(N, T, F_IN) for z in (mean, mn, mx, std)], axis=-1)
        ampl = (jnp.log(deg + 1.0) / ALOG)[:, :, None]
        atten = (ALOG / jnp.log(deg + 1.0))[:, :, None]

        Wp_x = lp['Wpost'][:, :F_IN, :]
        Wp_a = lp['Wpost'][:, F_IN:, :].reshape(T, 3, 4 * F_IN, F_OUT)
        def tower_mm(z, W):
            return jnp.einsum('ntf,tfo->nto', z, W)
        E1 = tower_mm(agg, Wp_a[:, 0])
        E2 = tower_mm(agg, Wp_a[:, 1])
        E3 = tower_mm(agg, Wp_a[:, 2])
        out = (jnp.einsum('nf,tfo->nto', h[:, :H], Wp_x)
               + E1 + E2 * ampl + E3 * atten + lp['bpost'])
        Wlin = _pad_to(_pad_to(lp['Wlin'], HP, 0), HP, 1)
        blin = _pad_to(lp['blin'], HP, 0)
        c = _matmul(out.reshape(N, T * F_OUT), Wlin[:T * F_OUT], blin, bm=400)

        mu = jnp.mean(c, axis=0)
        vv = jnp.mean(c * c, axis=0) - mu * mu
        gam = _pad_to(lp['bn_g'], HP, 0)
        bet = _pad_to(lp['bn_b'], HP, 0)
        cbn = (c - mu) / jnp.sqrt(vv + 1e-5) * gam + bet
        mask = (jnp.arange(HP) < H).astype(jnp.float32)
        h = h + jnp.maximum(cbn, 0.0) * mask / 2.0

        hs2 = h[src_s]
        hd2 = h[dst_s]
        W1 = jnp.concatenate([
            _pad_to(lp['eW1'][:H], HP, 0),
            _pad_to(lp['eW1'][H:2 * H], HP, 0),
            _pad_to(lp['eW1'][2 * H:], HP, 0)], axis=0)
        W1 = _pad_to(W1, HP, 1)
        z = _matmul(jnp.concatenate([hs2, hd2, e], axis=1), W1,
                    _pad_to(lp['eb1'], HP, 0), relu=True, bm=640)
        W2 = _pad_to(_pad_to(lp['eW2'], HP, 0), HP, 1)
        e = e + _matmul(z, W2, _pad_to(lp['eb2'], HP, 0), bm=640) / 2.0

    W1 = _pad_to(_pad_to(p['mlp_W1'], HP, 0), 64, 1)
    o = _matmul(h, W1, _pad_to(p['mlp_b1'], 64, 0), relu=True, bm=400)
    W2 = _pad_to(_pad_to(p['mlp_W2'], 64, 0), 32, 1)
    o = _matmul(o, W2, _pad_to(p['mlp_b2'], 32, 0), relu=True, bm=400)
    W3 = _pad_to(_pad_to(p['mlp_W3'], 32, 0), 8, 1)
    o = _matmul(o, W3, _pad_to(p['mlp_b3'], 8, 0), bm=400)
    return o[:, :2]


# speedup vs baseline: 19.7667x; 1.5868x over previous
import functools
import numpy as np
import jax
import jax.numpy as jnp
from jax import lax
from jax.experimental import pallas as pl
from jax.experimental.pallas import tpu as pltpu
from jax.experimental.pallas import tpu_sc as plsc

F_IN, T, F_OUT, H = 100, 5, 20, 100
ALOG = float(np.log(17.0))
F5 = T * F_IN
F5P = 512
HP = 128
D_E = 16

NC_SC, NS_SC = 2, 16
NW = NC_SC * NS_SC
NPW = 313
NP = NW * NPW
CE = 128
CHS = 128
EPW = 40 * CHS
EP = NW * EPW


def _sc_mesh():
    return plsc.VectorSubcoreMesh(core_axis_name="c", subcore_axis_name="s")


def _sc_hist(dst_pad):
    @functools.partial(
        pl.kernel, mesh=_sc_mesh(),
        out_type=jax.ShapeDtypeStruct((NW, 1, NP), jnp.int32),
        scratch_types=[
            pltpu.VMEM((1, NP), jnp.int32),
            pltpu.VMEM((EPW + 16,), jnp.int32),
        ],
    )
    def k(dst_hbm, out_hbm, hist_v, dbuf):
        wid = lax.axis_index("s") * NC_SC + lax.axis_index("c")

        def zbody(i, _):
            hist_v[0, pl.ds(i * 16, 16)] = jnp.zeros((16,), jnp.int32)
            return 0

        lax.fori_loop(0, NP // 16, zbody, 0)
        pltpu.sync_copy(dst_hbm.at[pl.ds(wid * EPW, EPW)],
                        dbuf.at[pl.ds(0, EPW)])
        iota = lax.iota(jnp.int32, 16)

        def ebody(i, _):
            d = dbuf[pl.ds(i, 16)][0]
            base = d & ~15
            lane = d - base
            win = hist_v[0, pl.ds(base, 16)]
            hist_v[0, pl.ds(base, 16)] = win + jnp.where(iota == lane, 1, 0)
            return 0

        lax.fori_loop(0, EPW, ebody, 0)
        pltpu.sync_copy(hist_v, out_hbm.at[wid])

    return k(dst_pad)


def _sc_scatter_sort(dst_pad, src_pad, e_rows, offs):
    @functools.partial(
        pl.kernel, mesh=_sc_mesh(),
        out_type=(jax.ShapeDtypeStruct((EP,), jnp.int32),
                  jax.ShapeDtypeStruct((EP,), jnp.int32),
                  jax.ShapeDtypeStruct((EP, HP), jnp.float32)),
        scratch_types=[
            pltpu.VMEM((1, NP + 16), jnp.int32),
            pltpu.VMEM((EPW + 16,), jnp.int32),
            pltpu.VMEM((EPW + 16,), jnp.int32),
            pltpu.VMEM((CHS, HP), jnp.float32),
            pltpu.VMEM((CHS,), jnp.int32),
            pltpu.SemaphoreType.DMA,
        ],
    )
    def k(dst_hbm, src_hbm, er_hbm, offs_hbm, dsts_hbm, srcs_hbm, es_hbm,
          offs_v, dbuf, sbuf, ebuf, slotb, sem):
        wid = lax.axis_index("s") * NC_SC + lax.axis_index("c")
        pltpu.sync_copy(offs_hbm.at[wid], offs_v)
        pltpu.sync_copy(dst_hbm.at[pl.ds(wid * EPW, EPW)],
                        dbuf.at[pl.ds(0, EPW)])
        pltpu.sync_copy(src_hbm.at[pl.ds(wid * EPW, EPW)],
                        sbuf.at[pl.ds(0, EPW)])
        iota = lax.iota(jnp.int32, 16)

        def chunk(t, _):
            pltpu.sync_copy(er_hbm.at[pl.ds(wid * EPW + t * CHS, CHS)], ebuf)

            def ebody(i, _):
                ii = t * CHS + i
                d = dbuf[pl.ds(ii, 16)][0]
                slot = offs_v[0, pl.ds(d, 16)][0]
                base = d & ~15
                win = offs_v[0, pl.ds(base, 16)]
                offs_v[0, pl.ds(base, 16)] = win + jnp.where(
                    iota == (d - base), 1, 0)
                sb = i & ~15
                wslot = slotb[pl.ds(sb, 16)]
                slotb[pl.ds(sb, 16)] = jnp.where(iota == (i - sb), slot, wslot)
                return 0

            lax.fori_loop(0, CHS, ebody, 0)
            pltpu.async_copy(sbuf.at[pl.ds(t * CHS, CHS)],
                             srcs_hbm.at[slotb], sem).wait()
            pltpu.async_copy(dbuf.at[pl.ds(t * CHS, CHS)],
                             dsts_hbm.at[slotb], sem).wait()
            pltpu.async_copy(ebuf, es_hbm.at[slotb], sem).wait()
            return 0

        lax.fori_loop(0, EPW // CHS, chunk, 0)

    return k(dst_pad, src_pad, e_rows, offs)


def _sc_segment_stats(S, rs2d):
    E = S.shape[0]
    mesh = plsc.VectorSubcoreMesh(core_axis_name="c", subcore_axis_name="s")

    @functools.partial(
        pl.kernel, mesh=mesh,
        out_type=jax.ShapeDtypeStruct((NP, 4, F5P), jnp.float32),
        scratch_types=[
            pltpu.VMEM((1, 336), jnp.int32),
            pltpu.VMEM((CE, F5P), jnp.float32),
            pltpu.VMEM((4, F5P), jnp.float32),
        ],
    )
    def k(S_hbm, rs_hbm, out_hbm, rs_v, buf, acc):
        wid = lax.axis_index("s") * NC_SC + lax.axis_index("c")
        pltpu.sync_copy(rs_hbm.at[wid], rs_v)

        def node_body(n, loaded_q):
            rsw = rs_v[0, pl.ds(n, 16)]
            start = rsw[0]
            end = rsw[1]
            qmax = E // CE - 1
            q0 = jnp.minimum(lax.div(start, CE), qmax)
            q1 = jnp.minimum(lax.div(jnp.maximum(end - 1, start), CE), qmax)

            def chunk_body(q, lq):
                @pl.when(q != lq)
                def _():
                    base = pl.multiple_of(q * CE, CE)
                    pltpu.sync_copy(S_hbm.at[pl.ds(base, CE)], buf)

                lo = jnp.maximum(start, q * CE)
                hi = jnp.minimum(end, (q + 1) * CE)
                first = lo == start
                rel = q * CE
                for c in range(F5P // 16):
                    sl = pl.ds(c * 16, 16)
                    zero = jnp.zeros((16,), jnp.float32)
                    big = jnp.full((16,), 3.0e38, jnp.float32)
                    s0 = jnp.where(first, zero, acc[0, sl])
                    q0v = jnp.where(first, zero, acc[1, sl])
                    mn0 = jnp.where(first, big, acc[2, sl])
                    mx0 = jnp.where(first, -big, acc[3, sl])

                    def ebody(i, st4):
                        s, sq, mn, mx = st4
                        v = buf[i - rel, sl]
                        return (s + v, sq + v * v,
                                jnp.minimum(mn, v), jnp.maximum(mx, v))

                    s, sq, mn, mx = lax.fori_loop(lo, hi, ebody,
                                                  (s0, q0v, mn0, mx0))
                    acc[0, sl] = s
                    acc[1, sl] = sq
                    acc[2, sl] = mn
                    acc[3, sl] = mx

                @pl.when(hi == end)
                def _():
                    pltpu.sync_copy(acc, out_hbm.at[wid * NPW + n])

                return q

            return lax.fori_loop(q0, q1 + 1, chunk_body, loaded_q)

        lax.fori_loop(0, NPW, node_body, jnp.int32(-1))

    return k(S, rs2d)


def _pad_to(a, n, axis):
    pad = n - a.shape[axis]
    if pad <= 0:
        return a
    cfg = [(0, 0)] * a.ndim
    cfg[axis] = (0, pad)
    return jnp.pad(a, cfg)


def _mm_body(x_ref, w_ref, b_ref, o_ref, *, relu):
    acc = jnp.dot(x_ref[...], w_ref[...], preferred_element_type=jnp.float32)
    acc = acc + b_ref[...]
    if relu:
        acc = jnp.maximum(acc, 0.0)
    o_ref[...] = acc


def _matmul(x, w, b, relu=False, bm=512):
    M, K = x.shape
    Nc = w.shape[1]
    assert M % bm == 0, (M, bm)
    grid = (M // bm,)
    return pl.pallas_call(
        functools.partial(_mm_body, relu=relu),
        grid=grid,
        in_specs=[
            pl.BlockSpec((bm, K), lambda i: (i, 0)),
            pl.BlockSpec((K, Nc), lambda i: (0, 0)),
            pl.BlockSpec((1, Nc), lambda i: (0, 0)),
        ],
        out_specs=pl.BlockSpec((bm, Nc), lambda i: (i, 0)),
        out_shape=jax.ShapeDtypeStruct((M, Nc), jnp.float32),
    )(x, w, b.reshape(1, Nc))


def _mmm_body(*refs, nx, relu):
    x_refs = refs[:nx]
    w_refs = refs[nx:2 * nx]
    b_ref = refs[2 * nx]
    o_ref = refs[2 * nx + 1]
    acc = b_ref[...]
    for xr, wr in zip(x_refs, w_refs):
        acc = acc + jnp.dot(xr[...], wr[...],
                            preferred_element_type=jnp.float32)
    if relu:
        acc = jnp.maximum(acc, 0.0)
    o_ref[...] = acc


def _matmul_multi(xs, ws, b, relu=False, bm=512):
    M = xs[0].shape[0]
    Nc = ws[0].shape[1]
    assert M % bm == 0, (M, bm)
    nx = len(xs)
    in_specs = (
        [pl.BlockSpec((bm, x.shape[1]), lambda i: (i, 0)) for x in xs]
        + [pl.BlockSpec(w.shape, lambda i: (0, 0)) for w in ws]
        + [pl.BlockSpec((1, Nc), lambda i: (0, 0))]
    )
    return pl.pallas_call(
        functools.partial(_mmm_body, nx=nx, relu=relu),
        grid=(M // bm,),
        in_specs=in_specs,
        out_specs=pl.BlockSpec((bm, Nc), lambda i: (i, 0)),
        out_shape=jax.ShapeDtypeStruct((M, Nc), jnp.float32),
    )(*xs, *ws, b.reshape(1, Nc))


def kernel(x, edge_index, edge_attr, params):
    N = x.shape[0]
    E = edge_index.shape[1]
    src, dst = edge_index[0], edge_index[1]

    pad = EP - E
    dstp = jnp.concatenate([dst.astype(jnp.int32),
                            jnp.full((pad,), NP - 1, jnp.int32)])
    srcp = jnp.concatenate([src.astype(jnp.int32),
                            jnp.zeros((pad,), jnp.int32)])
    eap = jnp.concatenate([edge_attr, jnp.zeros((pad, D_E), jnp.float32)])
    hist = _sc_hist(dstp)[:, 0, :]
    row_total = hist.sum(axis=0)
    row_start = jnp.concatenate(
        [jnp.zeros((1,), jnp.int32),
         jnp.cumsum(row_total, dtype=jnp.int32)])
    excl = (jnp.cumsum(hist, axis=0) - hist).astype(jnp.int32)
    offs = _pad_to(row_start[None, :NP] + excl, NP + 16, 1).reshape(NW, 1, NP + 16)

    p0 = params
    edge_W0 = _pad_to(_pad_to(p0['edge_W'], 32, 0), HP, 1)
    edge_b0 = _pad_to(p0['edge_b'], HP, 0)
    e_orig = _matmul(_pad_to(eap, 32, 1), edge_W0, edge_b0, bm=640)

    dst_s, src_s, e = _sc_scatter_sort(dstp, srcp, e_orig, offs)
    src_g = jnp.minimum(src_s, N - 1)
    dst_g = jnp.minimum(dst_s, N - 1)
    cnt = row_total[:N].astype(jnp.float32)
    rs_pad = jnp.concatenate(
        [row_start, jnp.full((336 - 1,), EP, jnp.int32)])
    rs2d = jnp.stack(
        [lax.dynamic_slice(rs_pad, (w * NPW,), (336,)) for w in range(NW)]
    ).reshape(NW, 1, 336)

    p = params
    node_W = _pad_to(p['node_W'], HP, 1)
    node_b = _pad_to(p['node_b'], HP, 0)
    h = _matmul(x, node_W, node_b, bm=400)

    for lp in p['layers']:
        Wa = lp['Wpre'][:, :F_IN, :]
        Wb = lp['Wpre'][:, F_IN:2 * F_IN, :]
        Wc = lp['Wpre'][:, 2 * F_IN:, :]
        WeeC = jnp.einsum('hf,tfo->hto', lp['Wee'], Wc,
                          precision=lax.Precision.HIGHEST).reshape(H, F5)
        b_all = (jnp.einsum('f,tfo->to', lp['bee'], Wc) + lp['bpre']).reshape(F5)
        WaF = jnp.transpose(Wa, (1, 0, 2)).reshape(F_IN, F5)
        WbF = jnp.transpose(Wb, (1, 0, 2)).reshape(F_IN, F5)

        WA = _pad_to(_pad_to(WaF, HP, 0), F5P, 1)
        bA = _pad_to(b_all, F5P, 0)
        A = _matmul(h, WA, bA, bm=400)

        hs = h[src_g]
        WbP = _pad_to(_pad_to(WbF, HP, 0), F5P, 1)
        WcP = _pad_to(_pad_to(WeeC, HP, 0), F5P, 1)
        S = _matmul_multi([hs, e], [WbP, WcP],
                          jnp.zeros((F5P,), jnp.float32), bm=640)

        stats = _sc_segment_stats(S, rs2d)[:N]
        sumS, sumsqS, minS, maxS = (stats[:, i] for i in range(4))

        empty = (cnt == 0)[:, None]
        deg = jnp.clip(cnt, 1.0, None)[:, None]
        meanS = jnp.where(empty, 0.0, sumS / deg)
        mean = jnp.where(empty, 0.0, A + meanS)
        var = jnp.maximum(jnp.where(empty, 0.0, sumsqS / deg) - meanS * meanS, 0.0)
        std = jnp.sqrt(var + 1e-5)
        mn = jnp.where(empty, 0.0, A + minS)
        mx = jnp.where(empty, 0.0, A + maxS)

        ampl = jnp.log(deg + 1.0) / ALOG
        atten = ALOG / jnp.log(deg + 1.0)

        Wp_x = lp['Wpost'][:, :F_IN, :]
        Wp_a = lp['Wpost'][:, F_IN:, :].reshape(T, 3, 4, F_IN, F_OUT)
        eyeT = jnp.eye(T, dtype=jnp.float32)

        def blkdiag(w):
            b = jnp.einsum('tfo,tu->tfuo', w, eyeT).reshape(F5, T * F_OUT)
            return _pad_to(_pad_to(b, F5P, 0), HP, 1)

        Wpost_blocks = [[blkdiag(Wp_a[:, k, s]) for s in range(4)]
                        for k in range(3)]
        WpxF = _pad_to(_pad_to(
            jnp.transpose(Wp_x, (1, 0, 2)).reshape(F_IN, T * F_OUT),
            HP, 0), HP, 1)
        bpost = _pad_to(lp['bpost'].reshape(T * F_OUT), HP, 0)
        stats4 = [mean, mn, mx, std]
        U1 = _matmul_multi(stats4 + [h], Wpost_blocks[0] + [WpxF],
                           bpost, bm=400)
        zb = jnp.zeros((HP,), jnp.float32)
        U2 = _matmul_multi(stats4, Wpost_blocks[1], zb, bm=400)
        U3 = _matmul_multi(stats4, Wpost_blocks[2], zb, bm=400)
        out = U1 + U2 * ampl + U3 * atten
        Wlin = _pad_to(_pad_to(lp['Wlin'], HP, 0), HP, 1)
        blin = _pad_to(lp['blin'], HP, 0)
        c = _matmul(out, Wlin, blin, bm=400)

        mu = jnp.mean(c, axis=0)
        vv = jnp.mean(c * c, axis=0) - mu * mu
        gam = _pad_to(lp['bn_g'], HP, 0)
        bet = _pad_to(lp['bn_b'], HP, 0)
        cbn = (c - mu) / jnp.sqrt(vv + 1e-5) * gam + bet
        mask = (jnp.arange(HP) < H).astype(jnp.float32)
        h = h + jnp.maximum(cbn, 0.0) * mask / 2.0

        hs2 = h[src_g]
        hd2 = h[dst_g]
        W1a = _pad_to(_pad_to(lp['eW1'][:H], HP, 0), HP, 1)
        W1b = _pad_to(_pad_to(lp['eW1'][H:2 * H], HP, 0), HP, 1)
        W1c = _pad_to(_pad_to(lp['eW1'][2 * H:], HP, 0), HP, 1)
        z = _matmul_multi([hs2, hd2, e], [W1a, W1b, W1c],
                          _pad_to(lp['eb1'], HP, 0), relu=True, bm=640)
        W2 = _pad_to(_pad_to(lp['eW2'], HP, 0), HP, 1)
        e = e + _matmul(z, W2, _pad_to(lp['eb2'], HP, 0), bm=640) / 2.0

    W1 = _pad_to(_pad_to(p['mlp_W1'], HP, 0), 64, 1)
    o = _matmul(h, W1, _pad_to(p['mlp_b1'], 64, 0), relu=True, bm=400)
    W2 = _pad_to(_pad_to(p['mlp_W2'], 64, 0), 32, 1)
    o = _matmul(o, W2, _pad_to(p['mlp_b2'], 32, 0), relu=True, bm=400)
    W3 = _pad_to(_pad_to(p['mlp_W3'], 32, 0), 8, 1)
    o = _matmul(o, W3, _pad_to(p['mlp_b3'], 8, 0), bm=400)
    return o[:, :2]


# speedup vs baseline: 26.0905x; 1.3199x over previous
import functools
import numpy as np
import jax
import jax.numpy as jnp
from jax import lax
from jax.experimental import pallas as pl
from jax.experimental.pallas import tpu as pltpu
from jax.experimental.pallas import tpu_sc as plsc

F_IN, T, F_OUT, H = 100, 5, 20, 100
ALOG = float(np.log(17.0))
F5 = T * F_IN
F5P = 512
HP = 128
D_E = 16

NC_SC, NS_SC = 2, 16
NW = NC_SC * NS_SC
NPW = 313
NP = NW * NPW
CE = 128
CHS = 128
EPW = 40 * CHS
EP = NW * EPW


def _sc_mesh():
    return plsc.VectorSubcoreMesh(core_axis_name="c", subcore_axis_name="s")


def _sc_hist(dst_pad):
    @functools.partial(
        pl.kernel, mesh=_sc_mesh(),
        out_type=jax.ShapeDtypeStruct((NW, 1, NP), jnp.int32),
        scratch_types=[
            pltpu.VMEM((1, NP), jnp.int32),
            pltpu.VMEM((EPW + 16,), jnp.int32),
        ],
    )
    def k(dst_hbm, out_hbm, hist_v, dbuf):
        wid = lax.axis_index("s") * NC_SC + lax.axis_index("c")

        def zbody(i, _):
            hist_v[0, pl.ds(i * 16, 16)] = jnp.zeros((16,), jnp.int32)
            return 0

        lax.fori_loop(0, NP // 16, zbody, 0)
        pltpu.sync_copy(dst_hbm.at[pl.ds(wid * EPW, EPW)],
                        dbuf.at[pl.ds(0, EPW)])
        iota = lax.iota(jnp.int32, 16)

        def ebody(i, _):
            d = dbuf[pl.ds(i, 16)][0]
            base = d & ~15
            lane = d - base
            win = hist_v[0, pl.ds(base, 16)]
            hist_v[0, pl.ds(base, 16)] = win + jnp.where(iota == lane, 1, 0)
            return 0

        lax.fori_loop(0, EPW, ebody, 0)
        pltpu.sync_copy(hist_v, out_hbm.at[wid])

    return k(dst_pad)


def _sc_scatter_sort(dst_pad, src_pad, e_rows, offs):
    @functools.partial(
        pl.kernel, mesh=_sc_mesh(),
        out_type=(jax.ShapeDtypeStruct((EP,), jnp.int32),
                  jax.ShapeDtypeStruct((EP,), jnp.int32),
                  jax.ShapeDtypeStruct((EP, HP), jnp.float32)),
        scratch_types=[
            pltpu.VMEM((1, NP + 16), jnp.int32),
            pltpu.VMEM((EPW + 16,), jnp.int32),
            pltpu.VMEM((EPW + 16,), jnp.int32),
            pltpu.VMEM((CHS, HP), jnp.float32),
            pltpu.VMEM((CHS,), jnp.int32),
            pltpu.SemaphoreType.DMA,
        ],
    )
    def k(dst_hbm, src_hbm, er_hbm, offs_hbm, dsts_hbm, srcs_hbm, es_hbm,
          offs_v, dbuf, sbuf, ebuf, slotb, sem):
        wid = lax.axis_index("s") * NC_SC + lax.axis_index("c")
        pltpu.sync_copy(offs_hbm.at[wid], offs_v)
        pltpu.sync_copy(dst_hbm.at[pl.ds(wid * EPW, EPW)],
                        dbuf.at[pl.ds(0, EPW)])
        pltpu.sync_copy(src_hbm.at[pl.ds(wid * EPW, EPW)],
                        sbuf.at[pl.ds(0, EPW)])
        iota = lax.iota(jnp.int32, 16)

        def chunk(t, _):
            pltpu.sync_copy(er_hbm.at[pl.ds(wid * EPW + t * CHS, CHS)], ebuf)

            def ebody(i, _):
                ii = t * CHS + i
                d = dbuf[pl.ds(ii, 16)][0]
                slot = offs_v[0, pl.ds(d, 16)][0]
                base = d & ~15
                win = offs_v[0, pl.ds(base, 16)]
                offs_v[0, pl.ds(base, 16)] = win + jnp.where(
                    iota == (d - base), 1, 0)
                sb = i & ~15
                wslot = slotb[pl.ds(sb, 16)]
                slotb[pl.ds(sb, 16)] = jnp.where(iota == (i - sb), slot, wslot)
                return 0

            lax.fori_loop(0, CHS, ebody, 0)
            pltpu.async_copy(sbuf.at[pl.ds(t * CHS, CHS)],
                             srcs_hbm.at[slotb], sem).wait()
            pltpu.async_copy(dbuf.at[pl.ds(t * CHS, CHS)],
                             dsts_hbm.at[slotb], sem).wait()
            pltpu.async_copy(ebuf, es_hbm.at[slotb], sem).wait()
            return 0

        lax.fori_loop(0, EPW // CHS, chunk, 0)

    return k(dst_pad, src_pad, e_rows, offs)


def _sc_segment_stats(S, rs2d):
    E = S.shape[0]
    mesh = plsc.VectorSubcoreMesh(core_axis_name="c", subcore_axis_name="s")

    @functools.partial(
        pl.kernel, mesh=mesh,
        out_type=jax.ShapeDtypeStruct((NP, 4, F5P), jnp.float32),
        scratch_types=[
            pltpu.VMEM((1, 336), jnp.int32),
            pltpu.VMEM((CE, F5P), jnp.float32),
            pltpu.VMEM((4, F5P), jnp.float32),
        ],
    )
    def k(S_hbm, rs_hbm, out_hbm, rs_v, buf, acc):
        wid = lax.axis_index("s") * NC_SC + lax.axis_index("c")
        pltpu.sync_copy(rs_hbm.at[wid], rs_v)

        def node_body(n, loaded_q):
            rsw = rs_v[0, pl.ds(n, 16)]
            start = rsw[0]
            end = jnp.where(wid * NPW + n < 10000, rsw[1], start)
            qmax = E // CE - 1
            q0 = jnp.minimum(lax.div(start, CE), qmax)
            q1 = jnp.minimum(lax.div(jnp.maximum(end - 1, start), CE), qmax)

            def chunk_body(q, lq):
                @pl.when(q != lq)
                def _():
                    base = pl.multiple_of(q * CE, CE)
                    pltpu.sync_copy(S_hbm.at[pl.ds(base, CE)], buf)

                lo = jnp.maximum(start, q * CE)
                hi = jnp.minimum(end, (q + 1) * CE)
                first = lo == start
                rel = q * CE
                for c in range(F5P // 32):
                    sl = pl.ds(c * 32, 16)
                    sl2 = pl.ds(c * 32 + 16, 16)
                    zero = jnp.zeros((16,), jnp.float32)
                    big = jnp.full((16,), 3.0e38, jnp.float32)
                    s0 = jnp.where(first, zero, acc[0, sl])
                    q0v = jnp.where(first, zero, acc[1, sl])
                    mn0 = jnp.where(first, big, acc[2, sl])
                    mx0 = jnp.where(first, -big, acc[3, sl])
                    s1 = jnp.where(first, zero, acc[0, sl2])
                    q1v = jnp.where(first, zero, acc[1, sl2])
                    mn1 = jnp.where(first, big, acc[2, sl2])
                    mx1 = jnp.where(first, -big, acc[3, sl2])

                    def ebody(i, st8):
                        s, sq, mn, mx, sb, sqb, mnb, mxb = st8
                        v = buf[i - rel, sl]
                        w = buf[i - rel, sl2]
                        return (s + v, sq + v * v,
                                jnp.minimum(mn, v), jnp.maximum(mx, v),
                                sb + w, sqb + w * w,
                                jnp.minimum(mnb, w), jnp.maximum(mxb, w))

                    (s, sq, mn, mx, sb, sqb, mnb, mxb) = lax.fori_loop(
                        lo, hi, ebody,
                        (s0, q0v, mn0, mx0, s1, q1v, mn1, mx1))
                    acc[0, sl] = s
                    acc[1, sl] = sq
                    acc[2, sl] = mn
                    acc[3, sl] = mx
                    acc[0, sl2] = sb
                    acc[1, sl2] = sqb
                    acc[2, sl2] = mnb
                    acc[3, sl2] = mxb

                @pl.when(hi == end)
                def _():
                    pltpu.sync_copy(acc, out_hbm.at[wid * NPW + n])

                return q

            return lax.fori_loop(q0, q1 + 1, chunk_body, loaded_q)

        lax.fori_loop(0, NPW, node_body, jnp.int32(-1))

    return k(S, rs2d)


def _pad_to(a, n, axis):
    pad = n - a.shape[axis]
    if pad <= 0:
        return a
    cfg = [(0, 0)] * a.ndim
    cfg[axis] = (0, pad)
    return jnp.pad(a, cfg)


def _mm_body(x_ref, w_ref, b_ref, o_ref, *, relu):
    acc = jnp.dot(x_ref[...], w_ref[...], preferred_element_type=jnp.float32)
    acc = acc + b_ref[...]
    if relu:
        acc = jnp.maximum(acc, 0.0)
    o_ref[...] = acc


def _matmul(x, w, b, relu=False, bm=512):
    M, K = x.shape
    Nc = w.shape[1]
    assert M % bm == 0, (M, bm)
    grid = (M // bm,)
    return pl.pallas_call(
        functools.partial(_mm_body, relu=relu),
        grid=grid,
        in_specs=[
            pl.BlockSpec((bm, K), lambda i: (i, 0)),
            pl.BlockSpec((K, Nc), lambda i: (0, 0)),
            pl.BlockSpec((1, Nc), lambda i: (0, 0)),
        ],
        out_specs=pl.BlockSpec((bm, Nc), lambda i: (i, 0)),
        out_shape=jax.ShapeDtypeStruct((M, Nc), jnp.float32),
    )(x, w, b.reshape(1, Nc))


def _mmm_body(*refs, nx, relu):
    x_refs = refs[:nx]
    w_refs = refs[nx:2 * nx]
    b_ref = refs[2 * nx]
    o_ref = refs[2 * nx + 1]
    acc = b_ref[...]
    for xr, wr in zip(x_refs, w_refs):
        acc = acc + jnp.dot(xr[...], wr[...],
                            preferred_element_type=jnp.float32)
    if relu:
        acc = jnp.maximum(acc, 0.0)
    o_ref[...] = acc


def _matmul_multi(xs, ws, b, relu=False, bm=512):
    M = xs[0].shape[0]
    Nc = ws[0].shape[1]
    assert M % bm == 0, (M, bm)
    nx = len(xs)
    in_specs = (
        [pl.BlockSpec((bm, x.shape[1]), lambda i: (i, 0)) for x in xs]
        + [pl.BlockSpec(w.shape, lambda i: (0, 0)) for w in ws]
        + [pl.BlockSpec((1, Nc), lambda i: (0, 0))]
    )
    return pl.pallas_call(
        functools.partial(_mmm_body, nx=nx, relu=relu),
        grid=(M // bm,),
        in_specs=in_specs,
        out_specs=pl.BlockSpec((bm, Nc), lambda i: (i, 0)),
        out_shape=jax.ShapeDtypeStruct((M, Nc), jnp.float32),
    )(*xs, *ws, b.reshape(1, Nc))


def kernel(x, edge_index, edge_attr, params):
    N = x.shape[0]
    E = edge_index.shape[1]
    src, dst = edge_index[0], edge_index[1]

    pad = EP - E
    dstp = jnp.concatenate([dst.astype(jnp.int32),
                            jnp.full((pad,), NP - 1, jnp.int32)])
    srcp = jnp.concatenate([src.astype(jnp.int32),
                            jnp.zeros((pad,), jnp.int32)])
    eap = jnp.concatenate([edge_attr, jnp.zeros((pad, D_E), jnp.float32)])
    hist = _sc_hist(dstp)[:, 0, :]
    row_total = hist.sum(axis=0)
    row_start = jnp.concatenate(
        [jnp.zeros((1,), jnp.int32),
         jnp.cumsum(row_total, dtype=jnp.int32)])
    excl = (jnp.cumsum(hist, axis=0) - hist).astype(jnp.int32)
    offs = _pad_to(row_start[None, :NP] + excl, NP + 16, 1).reshape(NW, 1, NP + 16)

    p0 = params
    edge_W0 = _pad_to(_pad_to(p0['edge_W'], 32, 0), HP, 1)
    edge_b0 = _pad_to(p0['edge_b'], HP, 0)
    e_orig = _matmul(_pad_to(eap, 32, 1), edge_W0, edge_b0, bm=640)

    dst_s, src_s, e = _sc_scatter_sort(dstp, srcp, e_orig, offs)
    src_g = jnp.minimum(src_s, N - 1)
    dst_g = jnp.minimum(dst_s, N - 1)
    cnt = row_total[:N].astype(jnp.float32)
    rs_pad = jnp.concatenate(
        [row_start, jnp.full((336 - 1,), EP, jnp.int32)])
    rs2d = jnp.stack(
        [lax.dynamic_slice(rs_pad, (w * NPW,), (336,)) for w in range(NW)]
    ).reshape(NW, 1, 336)

    p = params
    node_W = _pad_to(p['node_W'], HP, 1)
    node_b = _pad_to(p['node_b'], HP, 0)
    h = _matmul(x, node_W, node_b, bm=400)

    for lp in p['layers']:
        Wa = lp['Wpre'][:, :F_IN, :]
        Wb = lp['Wpre'][:, F_IN:2 * F_IN, :]
        Wc = lp['Wpre'][:, 2 * F_IN:, :]
        WeeC = jnp.einsum('hf,tfo->hto', lp['Wee'], Wc,
                          precision=lax.Precision.HIGHEST).reshape(H, F5)
        b_all = (jnp.einsum('f,tfo->to', lp['bee'], Wc) + lp['bpre']).reshape(F5)
        WaF = jnp.transpose(Wa, (1, 0, 2)).reshape(F_IN, F5)
        WbF = jnp.transpose(Wb, (1, 0, 2)).reshape(F_IN, F5)

        WA = _pad_to(_pad_to(WaF, HP, 0), F5P, 1)
        bA = _pad_to(b_all, F5P, 0)
        A = _matmul(h, WA, bA, bm=400)

        hs = h[src_g]
        WbP = _pad_to(_pad_to(WbF, HP, 0), F5P, 1)
        WcP = _pad_to(_pad_to(WeeC, HP, 0), F5P, 1)
        S = _matmul_multi([hs, e], [WbP, WcP],
                          jnp.zeros((F5P,), jnp.float32), bm=640)

        stats = _sc_segment_stats(S, rs2d)[:N]
        sumS, sumsqS, minS, maxS = (stats[:, i] for i in range(4))

        empty = (cnt == 0)[:, None]
        deg = jnp.clip(cnt, 1.0, None)[:, None]
        meanS = jnp.where(empty, 0.0, sumS / deg)
        mean = jnp.where(empty, 0.0, A + meanS)
        var = jnp.maximum(jnp.where(empty, 0.0, sumsqS / deg) - meanS * meanS, 0.0)
        std = jnp.sqrt(var + 1e-5)
        mn = jnp.where(empty, 0.0, A + minS)
        mx = jnp.where(empty, 0.0, A + maxS)

        ampl = jnp.log(deg + 1.0) / ALOG
        atten = ALOG / jnp.log(deg + 1.0)

        Wp_x = lp['Wpost'][:, :F_IN, :]
        Wp_a = lp['Wpost'][:, F_IN:, :].reshape(T, 3, 4, F_IN, F_OUT)
        eyeT = jnp.eye(T, dtype=jnp.float32)

        def blkdiag(w):
            b = jnp.einsum('tfo,tu->tfuo', w, eyeT).reshape(F5, T * F_OUT)
            return _pad_to(_pad_to(b, F5P, 0), HP, 1)

        Wpost_blocks = [[blkdiag(Wp_a[:, k, s]) for s in range(4)]
                        for k in range(3)]
        WpxF = _pad_to(_pad_to(
            jnp.transpose(Wp_x, (1, 0, 2)).reshape(F_IN, T * F_OUT),
            HP, 0), HP, 1)
        bpost = _pad_to(lp['bpost'].reshape(T * F_OUT), HP, 0)
        stats4 = [mean, mn, mx, std]
        U1 = _matmul_multi(stats4 + [h], Wpost_blocks[0] + [WpxF],
                           bpost, bm=400)
        zb = jnp.zeros((HP,), jnp.float32)
        U2 = _matmul_multi(stats4, Wpost_blocks[1], zb, bm=400)
        U3 = _matmul_multi(stats4, Wpost_blocks[2], zb, bm=400)
        out = U1 + U2 * ampl + U3 * atten
        Wlin = _pad_to(_pad_to(lp['Wlin'], HP, 0), HP, 1)
        blin = _pad_to(lp['blin'], HP, 0)
        c = _matmul(out, Wlin, blin, bm=400)

        mu = jnp.mean(c, axis=0)
        vv = jnp.mean(c * c, axis=0) - mu * mu
        gam = _pad_to(lp['bn_g'], HP, 0)
        bet = _pad_to(lp['bn_b'], HP, 0)
        cbn = (c - mu) / jnp.sqrt(vv + 1e-5) * gam + bet
        mask = (jnp.arange(HP) < H).astype(jnp.float32)
        h = h + jnp.maximum(cbn, 0.0) * mask / 2.0

        hs2 = h[src_g]
        hd2 = h[dst_g]
        W1a = _pad_to(_pad_to(lp['eW1'][:H], HP, 0), HP, 1)
        W1b = _pad_to(_pad_to(lp['eW1'][H:2 * H], HP, 0), HP, 1)
        W1c = _pad_to(_pad_to(lp['eW1'][2 * H:], HP, 0), HP, 1)
        z = _matmul_multi([hs2, hd2, e], [W1a, W1b, W1c],
                          _pad_to(lp['eb1'], HP, 0), relu=True, bm=640)
        W2 = _pad_to(_pad_to(lp['eW2'], HP, 0), HP, 1)
        e = e + _matmul(z, W2, _pad_to(lp['eb2'], HP, 0), bm=640) / 2.0

    W1 = _pad_to(_pad_to(p['mlp_W1'], HP, 0), 64, 1)
    o = _matmul(h, W1, _pad_to(p['mlp_b1'], 64, 0), relu=True, bm=400)
    W2 = _pad_to(_pad_to(p['mlp_W2'], 64, 0), 32, 1)
    o = _matmul(o, W2, _pad_to(p['mlp_b2'], 32, 0), relu=True, bm=400)
    W3 = _pad_to(_pad_to(p['mlp_W3'], 32, 0), 8, 1)
    o = _matmul(o, W3, _pad_to(p['mlp_b3'], 8, 0), bm=400)
    return o[:, :2]


# speedup vs baseline: 28.1037x; 1.0772x over previous
import functools
import numpy as np
import jax
import jax.numpy as jnp
from jax import lax
from jax.experimental import pallas as pl
from jax.experimental.pallas import tpu as pltpu
from jax.experimental.pallas import tpu_sc as plsc

F_IN, T, F_OUT, H = 100, 5, 20, 100
ALOG = float(np.log(17.0))
F5 = T * F_IN
F5P = 512
HP = 128
D_E = 16

NC_SC, NS_SC = 2, 16
NW = NC_SC * NS_SC
NPW = 313
NP = NW * NPW
CE = 128
CHS = 128
EPW = 40 * CHS
EP = NW * EPW


def _sc_mesh():
    return plsc.VectorSubcoreMesh(core_axis_name="c", subcore_axis_name="s")


def _sc_hist(dst_pad):
    @functools.partial(
        pl.kernel, mesh=_sc_mesh(),
        out_type=jax.ShapeDtypeStruct((NW, 1, NP), jnp.int32),
        scratch_types=[
            pltpu.VMEM((1, NP), jnp.int32),
            pltpu.VMEM((EPW + 16,), jnp.int32),
        ],
    )
    def k(dst_hbm, out_hbm, hist_v, dbuf):
        wid = lax.axis_index("s") * NC_SC + lax.axis_index("c")

        def zbody(i, _):
            hist_v[0, pl.ds(i * 16, 16)] = jnp.zeros((16,), jnp.int32)
            return 0

        lax.fori_loop(0, NP // 16, zbody, 0)
        pltpu.sync_copy(dst_hbm.at[pl.ds(wid * EPW, EPW)],
                        dbuf.at[pl.ds(0, EPW)])
        iota = lax.iota(jnp.int32, 16)

        def ebody(i, _):
            d = dbuf[pl.ds(i, 16)][0]
            base = d & ~15
            lane = d - base
            win = hist_v[0, pl.ds(base, 16)]
            hist_v[0, pl.ds(base, 16)] = win + jnp.where(iota == lane, 1, 0)
            return 0

        lax.fori_loop(0, EPW, ebody, 0)
        pltpu.sync_copy(hist_v, out_hbm.at[wid])

    return k(dst_pad)


def _sc_scatter_sort(dst_pad, src_pad, e_rows, offs):
    @functools.partial(
        pl.kernel, mesh=_sc_mesh(),
        out_type=(jax.ShapeDtypeStruct((EP,), jnp.int32),
                  jax.ShapeDtypeStruct((EP,), jnp.int32),
                  jax.ShapeDtypeStruct((EP, HP), jnp.float32)),
        scratch_types=[
            pltpu.VMEM((1, NP + 16), jnp.int32),
            pltpu.VMEM((EPW + 16,), jnp.int32),
            pltpu.VMEM((EPW + 16,), jnp.int32),
            pltpu.VMEM((CHS, HP), jnp.float32),
            pltpu.VMEM((CHS,), jnp.int32),
            pltpu.SemaphoreType.DMA,
        ],
    )
    def k(dst_hbm, src_hbm, er_hbm, offs_hbm, dsts_hbm, srcs_hbm, es_hbm,
          offs_v, dbuf, sbuf, ebuf, slotb, sem):
        wid = lax.axis_index("s") * NC_SC + lax.axis_index("c")
        pltpu.sync_copy(offs_hbm.at[wid], offs_v)
        pltpu.sync_copy(dst_hbm.at[pl.ds(wid * EPW, EPW)],
                        dbuf.at[pl.ds(0, EPW)])
        pltpu.sync_copy(src_hbm.at[pl.ds(wid * EPW, EPW)],
                        sbuf.at[pl.ds(0, EPW)])
        iota = lax.iota(jnp.int32, 16)

        def chunk(t, _):
            pltpu.sync_copy(er_hbm.at[pl.ds(wid * EPW + t * CHS, CHS)], ebuf)

            def ebody(i, _):
                ii = t * CHS + i
                d = dbuf[pl.ds(ii, 16)][0]
                slot = offs_v[0, pl.ds(d, 16)][0]
                base = d & ~15
                win = offs_v[0, pl.ds(base, 16)]
                offs_v[0, pl.ds(base, 16)] = win + jnp.where(
                    iota == (d - base), 1, 0)
                sb = i & ~15
                wslot = slotb[pl.ds(sb, 16)]
                slotb[pl.ds(sb, 16)] = jnp.where(iota == (i - sb), slot, wslot)
                return 0

            lax.fori_loop(0, CHS, ebody, 0)
            c1 = pltpu.async_copy(sbuf.at[pl.ds(t * CHS, CHS)],
                                  srcs_hbm.at[slotb], sem)
            c2 = pltpu.async_copy(dbuf.at[pl.ds(t * CHS, CHS)],
                                  dsts_hbm.at[slotb], sem)
            c3 = pltpu.async_copy(ebuf, es_hbm.at[slotb], sem)
            c1.wait()
            c2.wait()
            c3.wait()
            return 0

        lax.fori_loop(0, EPW // CHS, chunk, 0)

    return k(dst_pad, src_pad, e_rows, offs)


def _sc_segment_stats(S, rs2d):
    E = S.shape[0]
    mesh = plsc.VectorSubcoreMesh(core_axis_name="c", subcore_axis_name="s")

    @functools.partial(
        pl.kernel, mesh=mesh,
        out_type=jax.ShapeDtypeStruct((NP, 4, F5P), jnp.float32),
        scratch_types=[
            pltpu.VMEM((1, 336), jnp.int32),
            pltpu.VMEM((CE, F5P), jnp.float32),
            pltpu.VMEM((4, F5P), jnp.float32),
        ],
    )
    def k(S_hbm, rs_hbm, out_hbm, rs_v, buf, acc):
        wid = lax.axis_index("s") * NC_SC + lax.axis_index("c")
        pltpu.sync_copy(rs_hbm.at[wid], rs_v)

        def node_body(n, loaded_q):
            rsw = rs_v[0, pl.ds(n, 16)]
            start = rsw[0]
            end = jnp.where(wid * NPW + n < 10000, rsw[1], start)
            qmax = E // CE - 1
            q0 = jnp.minimum(lax.div(start, CE), qmax)
            q1 = jnp.minimum(lax.div(jnp.maximum(end - 1, start), CE), qmax)

            def chunk_body(q, lq):
                @pl.when(q != lq)
                def _():
                    base = pl.multiple_of(q * CE, CE)
                    pltpu.sync_copy(S_hbm.at[pl.ds(base, CE)], buf)

                lo = jnp.maximum(start, q * CE)
                hi = jnp.minimum(end, (q + 1) * CE)
                first = lo == start
                rel = q * CE
                UN = 4
                for c in range(F5P // (16 * UN)):
                    sls = [pl.ds(c * 16 * UN + u * 16, 16) for u in range(UN)]
                    zero = jnp.zeros((16,), jnp.float32)
                    big = jnp.full((16,), 3.0e38, jnp.float32)
                    init = []
                    for sl in sls:
                        init += [jnp.where(first, zero, acc[0, sl]),
                                 jnp.where(first, zero, acc[1, sl]),
                                 jnp.where(first, big, acc[2, sl]),
                                 jnp.where(first, -big, acc[3, sl])]

                    def ebody(i, st):
                        out = []
                        for u, sl in enumerate(sls):
                            s, sq, mn, mx = st[4 * u:4 * u + 4]
                            v = buf[i - rel, sl]
                            out += [s + v, sq + v * v,
                                    jnp.minimum(mn, v), jnp.maximum(mx, v)]
                        return tuple(out)

                    st = lax.fori_loop(lo, hi, ebody, tuple(init))
                    for u, sl in enumerate(sls):
                        acc[0, sl] = st[4 * u]
                        acc[1, sl] = st[4 * u + 1]
                        acc[2, sl] = st[4 * u + 2]
                        acc[3, sl] = st[4 * u + 3]

                @pl.when(hi == end)
                def _():
                    pltpu.sync_copy(acc, out_hbm.at[wid * NPW + n])

                return q

            return lax.fori_loop(q0, q1 + 1, chunk_body, loaded_q)

        lax.fori_loop(0, NPW, node_body, jnp.int32(-1))

    return k(S, rs2d)


def _pad_to(a, n, axis):
    pad = n - a.shape[axis]
    if pad <= 0:
        return a
    cfg = [(0, 0)] * a.ndim
    cfg[axis] = (0, pad)
    return jnp.pad(a, cfg)


def _mm_body(x_ref, w_ref, b_ref, o_ref, *, relu):
    acc = jnp.dot(x_ref[...], w_ref[...], preferred_element_type=jnp.float32)
    acc = acc + b_ref[...]
    if relu:
        acc = jnp.maximum(acc, 0.0)
    o_ref[...] = acc


def _matmul(x, w, b, relu=False, bm=512):
    M, K = x.shape
    Nc = w.shape[1]
    assert M % bm == 0, (M, bm)
    grid = (M // bm,)
    return pl.pallas_call(
        functools.partial(_mm_body, relu=relu),
        grid=grid,
        in_specs=[
            pl.BlockSpec((bm, K), lambda i: (i, 0)),
            pl.BlockSpec((K, Nc), lambda i: (0, 0)),
            pl.BlockSpec((1, Nc), lambda i: (0, 0)),
        ],
        out_specs=pl.BlockSpec((bm, Nc), lambda i: (i, 0)),
        out_shape=jax.ShapeDtypeStruct((M, Nc), jnp.float32),
    )(x, w, b.reshape(1, Nc))


def _mmm_body(*refs, nx, relu):
    x_refs = refs[:nx]
    w_refs = refs[nx:2 * nx]
    b_ref = refs[2 * nx]
    o_ref = refs[2 * nx + 1]
    acc = b_ref[...]
    for xr, wr in zip(x_refs, w_refs):
        acc = acc + jnp.dot(xr[...], wr[...],
                            preferred_element_type=jnp.float32)
    if relu:
        acc = jnp.maximum(acc, 0.0)
    o_ref[...] = acc


def _matmul_multi(xs, ws, b, relu=False, bm=512):
    M = xs[0].shape[0]
    Nc = ws[0].shape[1]
    assert M % bm == 0, (M, bm)
    nx = len(xs)
    in_specs = (
        [pl.BlockSpec((bm, x.shape[1]), lambda i: (i, 0)) for x in xs]
        + [pl.BlockSpec(w.shape, lambda i: (0, 0)) for w in ws]
        + [pl.BlockSpec((1, Nc), lambda i: (0, 0))]
    )
    return pl.pallas_call(
        functools.partial(_mmm_body, nx=nx, relu=relu),
        grid=(M // bm,),
        in_specs=in_specs,
        out_specs=pl.BlockSpec((bm, Nc), lambda i: (i, 0)),
        out_shape=jax.ShapeDtypeStruct((M, Nc), jnp.float32),
    )(*xs, *ws, b.reshape(1, Nc))


def kernel(x, edge_index, edge_attr, params):
    N = x.shape[0]
    E = edge_index.shape[1]
    src, dst = edge_index[0], edge_index[1]

    pad = EP - E
    dstp = jnp.concatenate([dst.astype(jnp.int32),
                            jnp.full((pad,), NP - 1, jnp.int32)])
    srcp = jnp.concatenate([src.astype(jnp.int32),
                            jnp.zeros((pad,), jnp.int32)])
    eap = jnp.concatenate([edge_attr, jnp.zeros((pad, D_E), jnp.float32)])
    hist = _sc_hist(dstp)[:, 0, :]
    row_total = hist.sum(axis=0)
    row_start = jnp.concatenate(
        [jnp.zeros((1,), jnp.int32),
         jnp.cumsum(row_total, dtype=jnp.int32)])
    excl = (jnp.cumsum(hist, axis=0) - hist).astype(jnp.int32)
    offs = _pad_to(row_start[None, :NP] + excl, NP + 16, 1).reshape(NW, 1, NP + 16)

    p0 = params
    edge_W0 = _pad_to(_pad_to(p0['edge_W'], 32, 0), HP, 1)
    edge_b0 = _pad_to(p0['edge_b'], HP, 0)
    e_orig = _matmul(_pad_to(eap, 32, 1), edge_W0, edge_b0, bm=640)

    dst_s, src_s, e = _sc_scatter_sort(dstp, srcp, e_orig, offs)
    src_g = jnp.minimum(src_s, N - 1)
    dst_g = jnp.minimum(dst_s, N - 1)
    cnt = row_total[:N].astype(jnp.float32)
    rs_pad = jnp.concatenate(
        [row_start, jnp.full((336 - 1,), EP, jnp.int32)])
    rs2d = jnp.stack(
        [lax.dynamic_slice(rs_pad, (w * NPW,), (336,)) for w in range(NW)]
    ).reshape(NW, 1, 336)

    p = params
    node_W = _pad_to(p['node_W'], HP, 1)
    node_b = _pad_to(p['node_b'], HP, 0)
    h = _matmul(x, node_W, node_b, bm=400)

    for lp in p['layers']:
        Wa = lp['Wpre'][:, :F_IN, :]
        Wb = lp['Wpre'][:, F_IN:2 * F_IN, :]
        Wc = lp['Wpre'][:, 2 * F_IN:, :]
        WeeC = jnp.einsum('hf,tfo->hto', lp['Wee'], Wc,
                          precision=lax.Precision.HIGHEST).reshape(H, F5)
        b_all = (jnp.einsum('f,tfo->to', lp['bee'], Wc) + lp['bpre']).reshape(F5)
        WaF = jnp.transpose(Wa, (1, 0, 2)).reshape(F_IN, F5)
        WbF = jnp.transpose(Wb, (1, 0, 2)).reshape(F_IN, F5)

        WA = _pad_to(_pad_to(WaF, HP, 0), F5P, 1)
        bA = _pad_to(b_all, F5P, 0)
        A = _matmul(h, WA, bA, bm=400)

        hs = h[src_g]
        WbP = _pad_to(_pad_to(WbF, HP, 0), F5P, 1)
        WcP = _pad_to(_pad_to(WeeC, HP, 0), F5P, 1)
        S = _matmul_multi([hs, e], [WbP, WcP],
                          jnp.zeros((F5P,), jnp.float32), bm=640)

        stats = _sc_segment_stats(S, rs2d)[:N]
        sumS, sumsqS, minS, maxS = (stats[:, i] for i in range(4))

        empty = (cnt == 0)[:, None]
        deg = jnp.clip(cnt, 1.0, None)[:, None]
        meanS = jnp.where(empty, 0.0, sumS / deg)
        mean = jnp.where(empty, 0.0, A + meanS)
        var = jnp.maximum(jnp.where(empty, 0.0, sumsqS / deg) - meanS * meanS, 0.0)
        std = jnp.sqrt(var + 1e-5)
        mn = jnp.where(empty, 0.0, A + minS)
        mx = jnp.where(empty, 0.0, A + maxS)

        ampl = jnp.log(deg + 1.0) / ALOG
        atten = ALOG / jnp.log(deg + 1.0)

        Wp_x = lp['Wpost'][:, :F_IN, :]
        Wp_a = lp['Wpost'][:, F_IN:, :].reshape(T, 3, 4, F_IN, F_OUT)
        eyeT = jnp.eye(T, dtype=jnp.float32)

        def blkdiag(w):
            b = jnp.einsum('tfo,tu->tfuo', w, eyeT).reshape(F5, T * F_OUT)
            return _pad_to(_pad_to(b, F5P, 0), HP, 1)

        Wpost_blocks = [[blkdiag(Wp_a[:, k, s]) for s in range(4)]
                        for k in range(3)]
        WpxF = _pad_to(_pad_to(
            jnp.transpose(Wp_x, (1, 0, 2)).reshape(F_IN, T * F_OUT),
            HP, 0), HP, 1)
        bpost = _pad_to(lp['bpost'].reshape(T * F_OUT), HP, 0)
        stats4 = [mean, mn, mx, std]
        U1 = _matmul_multi(stats4 + [h], Wpost_blocks[0] + [WpxF],
                           bpost, bm=400)
        zb = jnp.zeros((HP,), jnp.float32)
        U2 = _matmul_multi(stats4, Wpost_blocks[1], zb, bm=400)
        U3 = _matmul_multi(stats4, Wpost_blocks[2], zb, bm=400)
        out = U1 + U2 * ampl + U3 * atten
        Wlin = _pad_to(_pad_to(lp['Wlin'], HP, 0), HP, 1)
        blin = _pad_to(lp['blin'], HP, 0)
        c = _matmul(out, Wlin, blin, bm=400)

        mu = jnp.mean(c, axis=0)
        vv = jnp.mean(c * c, axis=0) - mu * mu
        gam = _pad_to(lp['bn_g'], HP, 0)
        bet = _pad_to(lp['bn_b'], HP, 0)
        cbn = (c - mu) / jnp.sqrt(vv + 1e-5) * gam + bet
        mask = (jnp.arange(HP) < H).astype(jnp.float32)
        h = h + jnp.maximum(cbn, 0.0) * mask / 2.0

        hs2 = h[src_g]
        hd2 = h[dst_g]
        W1a = _pad_to(_pad_to(lp['eW1'][:H], HP, 0), HP, 1)
        W1b = _pad_to(_pad_to(lp['eW1'][H:2 * H], HP, 0), HP, 1)
        W1c = _pad_to(_pad_to(lp['eW1'][2 * H:], HP, 0), HP, 1)
        z = _matmul_multi([hs2, hd2, e], [W1a, W1b, W1c],
                          _pad_to(lp['eb1'], HP, 0), relu=True, bm=640)
        W2 = _pad_to(_pad_to(lp['eW2'], HP, 0), HP, 1)
        e = e + _matmul(z, W2, _pad_to(lp['eb2'], HP, 0), bm=640) / 2.0

    W1 = _pad_to(_pad_to(p['mlp_W1'], HP, 0), 64, 1)
    o = _matmul(h, W1, _pad_to(p['mlp_b1'], 64, 0), relu=True, bm=400)
    W2 = _pad_to(_pad_to(p['mlp_W2'], 64, 0), 32, 1)
    o = _matmul(o, W2, _pad_to(p['mlp_b2'], 32, 0), relu=True, bm=400)
    W3 = _pad_to(_pad_to(p['mlp_W3'], 32, 0), 8, 1)
    o = _matmul(o, W3, _pad_to(p['mlp_b3'], 8, 0), bm=400)
    return o[:, :2]
